# Initial kernel scaffold; baseline (speedup 1.0000x reference)
import jax
import jax.numpy as jnp
from jax.experimental import pallas as pl


def kernel(feature1, xyz1, query_xyz, error, shift_chose, knn_num, prefix_W, prefix_b, pos_W1, pos_b1, pos_W2, pos_b2):
    raise NotImplementedError("write your pallas kernel here")


# speedup vs baseline: 10.0559x; 10.0559x over previous
import functools

import jax
import jax.numpy as jnp
from jax import lax
from jax.experimental import pallas as pl
from jax.experimental.pallas import tpu as pltpu
from jax.experimental.pallas import tpu_sc as plsc

K = 16
TM = 256
TMC = 256
D_TAB = 128


def _knn_body(xyzT_ref, q_ref, idx_ref, flat_ref, dist_ref):
    b = pl.program_id(0)
    n = xyzT_ref.shape[2]
    x = xyzT_ref[0]
    q = q_ref[0]
    x2 = jnp.sum(x * x, axis=0, keepdims=True)
    q2 = jnp.sum(q * q, axis=1, keepdims=True)
    dots = jnp.dot(q.astype(jnp.bfloat16), x.astype(jnp.bfloat16),
                   preferred_element_type=jnp.float32)
    dist_ref[...] = (q2 + x2) - 2.0 * dots
    iota = lax.broadcasted_iota(jnp.int32, (TM, n), 1)
    for k in range(K):
        d = dist_ref[...]
        mn = jnp.min(d, axis=1, keepdims=True)
        cand = jnp.where(d == mn, iota, n)
        sel = jnp.min(cand, axis=1, keepdims=True)
        idx_ref[0, :, k:k + 1] = sel
        flat_ref[0, :, k:k + 1] = sel + b * n
        dist_ref[...] = jnp.where(iota == sel, jnp.inf, d)


def _knn_call(xyzT, query):
    B, _, N = xyzT.shape
    M = query.shape[1]
    grid = (B, M // TM)
    return pl.pallas_call(
        _knn_body,
        grid=grid,
        in_specs=[
            pl.BlockSpec((1, 3, N), lambda b, m: (b, 0, 0)),
            pl.BlockSpec((1, TM, 3), lambda b, m: (b, m, 0)),
        ],
        out_specs=[
            pl.BlockSpec((1, TM, K), lambda b, m: (b, m, 0)),
            pl.BlockSpec((1, TM, K), lambda b, m: (b, m, 0)),
        ],
        out_shape=[
            jax.ShapeDtypeStruct((B, M, K), jnp.int32),
            jax.ShapeDtypeStruct((B, M, K), jnp.int32),
        ],
        scratch_shapes=[pltpu.VMEM((TM, N), jnp.float32)],
    )(xyzT, query)


def _sc_gather(table, idx_flat):
    R = idx_flat.shape[0]
    info = plsc.get_sparse_core_info()
    NW = info.num_cores * info.num_subcores
    rows_per_w = R // NW
    CH = min(512, rows_per_w)
    mesh = plsc.VectorSubcoreMesh(core_axis_name="c", subcore_axis_name="s")

    @functools.partial(
        pl.kernel,
        mesh=mesh,
        out_type=jax.ShapeDtypeStruct((R, D_TAB), jnp.float32),
        scratch_types=[
            pltpu.VMEM((CH,), jnp.int32),
            pltpu.VMEM((CH, D_TAB), jnp.float32),
            pltpu.SemaphoreType.DMA,
        ],
    )
    def gk(table_hbm, idx_hbm, out_hbm, idx_v, rows_v, sem):
        wid = lax.axis_index("s") * info.num_cores + lax.axis_index("c")
        for c in range(rows_per_w // CH):
            base = wid * rows_per_w + c * CH
            pltpu.sync_copy(idx_hbm.at[pl.ds(base, CH)], idx_v)
            pltpu.async_copy(table_hbm.at[idx_v], rows_v, sem).wait()
            pltpu.sync_copy(rows_v, out_hbm.at[pl.ds(base, CH)])

    return gk(table, idx_flat)


def _mlp_body(g_ref, qrep_ref, pW_ref, pb_ref, W1_ref, b1_ref, W2_ref, b2_ref,
              out_ref):
    hi = jax.lax.Precision.HIGHEST
    g = g_ref[0]
    feat = g[:, :64]
    rel = g[:, 64:67] - qrep_ref[0]
    f = jnp.dot(feat, pW_ref[...], precision=hi,
                preferred_element_type=jnp.float32) + pb_ref[...]
    omega = jnp.exp(
        lax.broadcasted_iota(jnp.int32, (1, 10), 1).astype(jnp.float32)
        * (-jnp.log(10000.0) / 10.0))
    parts = []
    for i in range(3):
        arg = rel[:, i:i + 1] * omega
        parts.append(jnp.sin(arg))
        parts.append(jnp.cos(arg))
    emb = jnp.concatenate(parts, axis=1)
    h = jnp.maximum(
        jnp.dot(emb, W1_ref[...], precision=hi,
                preferred_element_type=jnp.float32) + b1_ref[...], 0.0)
    o = jnp.dot(h, W2_ref[...], precision=hi,
                preferred_element_type=jnp.float32) + b2_ref[...]
    gf = f + o
    g3 = gf.reshape(TMC, K, 64)
    w = jnp.sum(g3 * g3[:, 0:1, :], axis=-1)
    mx = jnp.max(w, axis=-1, keepdims=True)
    e = jnp.exp(w - mx)
    sm = e / jnp.sum(e, axis=-1, keepdims=True) * 0.4
    kio = lax.broadcasted_iota(jnp.int32, (TMC, K), 1)
    out_ref[0] = sm + jnp.where(kio == 0, 0.6, 0.0)


def _mlp_call(g, qrep, pW, pb, W1, b1, W2, b2):
    B = g.shape[0]
    MK = g.shape[1]
    M = MK // K
    grid = (B, M // TMC)
    return pl.pallas_call(
        _mlp_body,
        grid=grid,
        in_specs=[
            pl.BlockSpec((1, TMC * K, D_TAB), lambda b, m: (b, m, 0)),
            pl.BlockSpec((1, TMC * K, 3), lambda b, m: (b, m, 0)),
            pl.BlockSpec((64, 64), lambda b, m: (0, 0)),
            pl.BlockSpec((1, 64), lambda b, m: (0, 0)),
            pl.BlockSpec((60, 64), lambda b, m: (0, 0)),
            pl.BlockSpec((1, 64), lambda b, m: (0, 0)),
            pl.BlockSpec((64, 64), lambda b, m: (0, 0)),
            pl.BlockSpec((1, 64), lambda b, m: (0, 0)),
        ],
        out_specs=pl.BlockSpec((1, TMC, K), lambda b, m: (b, m, 0)),
        out_shape=jax.ShapeDtypeStruct((B, M, K), jnp.float32),
    )(g, qrep, pW, pb, W1, b1, W2, b2)


def kernel(feature1, xyz1, query_xyz, error, shift_chose, knn_num,
           prefix_W, prefix_b, pos_W1, pos_b1, pos_W2, pos_b2):
    B, N, FD = feature1.shape
    M = query_xyz.shape[1]
    xyzT = jnp.transpose(xyz1, (0, 2, 1))
    point_index, flat_idx = _knn_call(xyzT, query_xyz)
    table = jnp.concatenate(
        [feature1, xyz1, jnp.zeros((B, N, D_TAB - FD - 3), jnp.float32)],
        axis=-1).reshape(B * N, D_TAB)
    gathered = _sc_gather(table, flat_idx.reshape(-1))
    g = gathered.reshape(B, M * K, D_TAB)
    qrep = jnp.broadcast_to(query_xyz[:, :, None, :],
                            (B, M, K, 3)).reshape(B, M * K, 3)
    weight = _mlp_call(g, qrep, prefix_W, prefix_b.reshape(1, -1),
                       pos_W1, pos_b1.reshape(1, -1),
                       pos_W2, pos_b2.reshape(1, -1))
    return (weight, point_index)


# speedup vs baseline: 10.9883x; 1.0927x over previous
import functools

import jax
import jax.numpy as jnp
from jax import lax
from jax.experimental import pallas as pl
from jax.experimental.pallas import tpu as pltpu
from jax.experimental.pallas import tpu_sc as plsc

K = 16
TM = 256
TMC = 256
D_TAB = 128


def _knn_body(xyzT_ref, q_ref, idx_ref, flat_ref, dist_ref):
    b = pl.program_id(0)
    n = xyzT_ref.shape[2]
    x = xyzT_ref[0]
    q = q_ref[0]
    x2 = jnp.sum(x * x, axis=0, keepdims=True)
    q2 = jnp.sum(q * q, axis=1, keepdims=True)
    dots = jnp.dot(q.astype(jnp.bfloat16), x.astype(jnp.bfloat16),
                   preferred_element_type=jnp.float32)
    dist_ref[...] = (q2 + x2) - 2.0 * dots
    iota = lax.broadcasted_iota(jnp.int32, (TM, n), 1)
    for k in range(K):
        d = dist_ref[...]
        mn = jnp.min(d, axis=1, keepdims=True)
        cand = jnp.where(d == mn, iota, n)
        sel = jnp.min(cand, axis=1, keepdims=True)
        idx_ref[0, :, k:k + 1] = sel
        flat_ref[0, :, k:k + 1] = sel + b * n
        dist_ref[...] = jnp.where(iota == sel, jnp.inf, d)


def _knn_call(xyzT, query):
    B, _, N = xyzT.shape
    M = query.shape[1]
    grid = (B, M // TM)
    return pl.pallas_call(
        _knn_body,
        grid=grid,
        in_specs=[
            pl.BlockSpec((1, 3, N), lambda b, m: (b, 0, 0)),
            pl.BlockSpec((1, TM, 3), lambda b, m: (b, m, 0)),
        ],
        out_specs=[
            pl.BlockSpec((1, TM, K), lambda b, m: (b, m, 0)),
            pl.BlockSpec((1, TM, K), lambda b, m: (b, m, 0)),
        ],
        out_shape=[
            jax.ShapeDtypeStruct((B, M, K), jnp.int32),
            jax.ShapeDtypeStruct((B, M, K), jnp.int32),
        ],
        scratch_shapes=[pltpu.VMEM((TM, N), jnp.float32)],
    )(xyzT, query)


def _sc_gather(table, idx_flat):
    R = idx_flat.shape[0]
    info = plsc.get_sparse_core_info()
    NW = info.num_cores * info.num_subcores
    rows_per_w = R // NW
    CH = min(512, rows_per_w)
    mesh = plsc.VectorSubcoreMesh(core_axis_name="c", subcore_axis_name="s")

    @functools.partial(
        pl.kernel,
        mesh=mesh,
        out_type=jax.ShapeDtypeStruct((R, D_TAB), jnp.float32),
        scratch_types=[
            pltpu.VMEM((CH,), jnp.int32),
            pltpu.VMEM((CH, D_TAB), jnp.float32),
            pltpu.SemaphoreType.DMA,
        ],
    )
    def gk(table_hbm, idx_hbm, out_hbm, idx_v, rows_v, sem):
        wid = lax.axis_index("s") * info.num_cores + lax.axis_index("c")
        for c in range(rows_per_w // CH):
            base = wid * rows_per_w + c * CH
            pltpu.sync_copy(idx_hbm.at[pl.ds(base, CH)], idx_v)
            pltpu.async_copy(table_hbm.at[idx_v], rows_v, sem).wait()
            pltpu.sync_copy(rows_v, out_hbm.at[pl.ds(base, CH)])

    return gk(table, idx_flat)


def _mlp_body(g_ref, qrep_ref, pW_ref, pb_ref, W1_ref, b1_ref, W2_ref, b2_ref,
              out_ref):
    bf = jnp.bfloat16
    g = g_ref[0]
    feat = g[:, :64]
    rel = g[:, 64:67] - qrep_ref[0]
    f = jnp.dot(feat.astype(bf), pW_ref[...].astype(bf),
                preferred_element_type=jnp.float32) + pb_ref[...]
    omega = jnp.exp(
        lax.broadcasted_iota(jnp.int32, (1, 10), 1).astype(jnp.float32)
        * (-jnp.log(10000.0) / 10.0))
    parts = []
    for i in range(3):
        arg = rel[:, i:i + 1] * omega
        parts.append(jnp.sin(arg))
        parts.append(jnp.cos(arg))
    emb = jnp.concatenate(parts, axis=1)
    h = jnp.maximum(
        jnp.dot(emb.astype(bf), W1_ref[...].astype(bf),
                preferred_element_type=jnp.float32) + b1_ref[...], 0.0)
    o = jnp.dot(h.astype(bf), W2_ref[...].astype(bf),
                preferred_element_type=jnp.float32) + b2_ref[...]
    gf = f + o
    g3 = gf.reshape(TMC, K, 64)
    w = jnp.sum(g3 * g3[:, 0:1, :], axis=-1)
    mx = jnp.max(w, axis=-1, keepdims=True)
    e = jnp.exp(w - mx)
    sm = e / jnp.sum(e, axis=-1, keepdims=True) * 0.4
    kio = lax.broadcasted_iota(jnp.int32, (TMC, K), 1)
    out_ref[0] = sm + jnp.where(kio == 0, 0.6, 0.0)


def _mlp_call(g, qrep, pW, pb, W1, b1, W2, b2):
    B = g.shape[0]
    MK = g.shape[1]
    M = MK // K
    grid = (B, M // TMC)
    return pl.pallas_call(
        _mlp_body,
        grid=grid,
        in_specs=[
            pl.BlockSpec((1, TMC * K, D_TAB), lambda b, m: (b, m, 0)),
            pl.BlockSpec((1, TMC * K, 3), lambda b, m: (b, m, 0)),
            pl.BlockSpec((64, 64), lambda b, m: (0, 0)),
            pl.BlockSpec((1, 64), lambda b, m: (0, 0)),
            pl.BlockSpec((60, 64), lambda b, m: (0, 0)),
            pl.BlockSpec((1, 64), lambda b, m: (0, 0)),
            pl.BlockSpec((64, 64), lambda b, m: (0, 0)),
            pl.BlockSpec((1, 64), lambda b, m: (0, 0)),
        ],
        out_specs=pl.BlockSpec((1, TMC, K), lambda b, m: (b, m, 0)),
        out_shape=jax.ShapeDtypeStruct((B, M, K), jnp.float32),
    )(g, qrep, pW, pb, W1, b1, W2, b2)


def kernel(feature1, xyz1, query_xyz, error, shift_chose, knn_num,
           prefix_W, prefix_b, pos_W1, pos_b1, pos_W2, pos_b2):
    B, N, FD = feature1.shape
    M = query_xyz.shape[1]
    xyzT = jnp.transpose(xyz1, (0, 2, 1))
    point_index, flat_idx = _knn_call(xyzT, query_xyz)
    table = jnp.concatenate(
        [feature1, xyz1, jnp.zeros((B, N, D_TAB - FD - 3), jnp.float32)],
        axis=-1).reshape(B * N, D_TAB)
    gathered = _sc_gather(table, flat_idx.reshape(-1))
    g = gathered.reshape(B, M * K, D_TAB)
    qrep = jnp.broadcast_to(query_xyz[:, :, None, :],
                            (B, M, K, 3)).reshape(B, M * K, 3)
    weight = _mlp_call(g, qrep, prefix_W, prefix_b.reshape(1, -1),
                       pos_W1, pos_b1.reshape(1, -1),
                       pos_W2, pos_b2.reshape(1, -1))
    return (weight, point_index)


# speedup vs baseline: 14.4511x; 1.3151x over previous
import functools

import jax
import jax.numpy as jnp
from jax import lax
from jax.experimental import pallas as pl
from jax.experimental.pallas import tpu as pltpu
from jax.experimental.pallas import tpu_sc as plsc

K = 16
TM = 256
TMC = 256
D_TAB = 128


def _knn_body(xyzT_ref, q_ref, idx_ref, flat_ref, dist_ref):
    b = pl.program_id(0)
    n = xyzT_ref.shape[2]
    x = xyzT_ref[0]
    q = q_ref[0]
    x2 = jnp.sum(x * x, axis=0, keepdims=True)
    q2 = jnp.sum(q * q, axis=1, keepdims=True)
    dots = jnp.dot(q.astype(jnp.bfloat16), x.astype(jnp.bfloat16),
                   preferred_element_type=jnp.float32)
    dist_ref[...] = (q2 + x2) - 2.0 * dots
    iota = lax.broadcasted_iota(jnp.int32, (TM, n), 1)
    for k in range(K):
        d = dist_ref[...]
        mn = jnp.min(d, axis=1, keepdims=True)
        cand = jnp.where(d == mn, iota, n)
        sel = jnp.min(cand, axis=1, keepdims=True)
        idx_ref[0, :, k:k + 1] = sel
        flat_ref[0, :, k:k + 1] = sel + b * n
        dist_ref[...] = jnp.where(iota == sel, jnp.inf, d)


def _knn_call(xyzT, query):
    B, _, N = xyzT.shape
    M = query.shape[1]
    grid = (B, M // TM)
    return pl.pallas_call(
        _knn_body,
        grid=grid,
        in_specs=[
            pl.BlockSpec((1, 3, N), lambda b, m: (b, 0, 0)),
            pl.BlockSpec((1, TM, 3), lambda b, m: (b, m, 0)),
        ],
        out_specs=[
            pl.BlockSpec((1, TM, K), lambda b, m: (b, m, 0)),
            pl.BlockSpec((1, TM, K), lambda b, m: (b, m, 0)),
        ],
        out_shape=[
            jax.ShapeDtypeStruct((B, M, K), jnp.int32),
            jax.ShapeDtypeStruct((B, M, K), jnp.int32),
        ],
        scratch_shapes=[pltpu.VMEM((TM, N), jnp.float32)],
    )(xyzT, query)


def _sc_gather(table, idx_flat):
    R = idx_flat.shape[0]
    info = plsc.get_sparse_core_info()
    NW = info.num_cores * info.num_subcores
    rows_per_w = R // NW
    CH = min(512, rows_per_w)
    mesh = plsc.VectorSubcoreMesh(core_axis_name="c", subcore_axis_name="s")

    @functools.partial(
        pl.kernel,
        mesh=mesh,
        out_type=jax.ShapeDtypeStruct((R, D_TAB), jnp.float32),
        scratch_types=[
            pltpu.VMEM((CH,), jnp.int32),
            pltpu.VMEM((CH, D_TAB), jnp.float32),
            pltpu.SemaphoreType.DMA,
        ],
    )
    def gk(table_hbm, idx_hbm, out_hbm, idx_v, rows_v, sem):
        wid = lax.axis_index("s") * info.num_cores + lax.axis_index("c")
        for c in range(rows_per_w // CH):
            base = wid * rows_per_w + c * CH
            pltpu.sync_copy(idx_hbm.at[pl.ds(base, CH)], idx_v)
            pltpu.async_copy(table_hbm.at[idx_v], rows_v, sem).wait()
            pltpu.sync_copy(rows_v, out_hbm.at[pl.ds(base, CH)])

    return gk(table, idx_flat)


def _mlp_body(g_ref, qrep_ref, pW_ref, pb_ref, W1_ref, b1_ref, W2_ref, b2_ref,
              out_ref):
    bf = jnp.bfloat16
    g = g_ref[0]
    feat = g[:, :64]
    rel = g[:, 64:67] - qrep_ref[0]
    f = jnp.dot(feat.astype(bf), pW_ref[...].astype(bf),
                preferred_element_type=jnp.float32) + pb_ref[...]
    rows = rel.shape[0]
    lane60 = lax.broadcasted_iota(jnp.int32, (1, 60), 1)
    m20 = lane60 % 20
    omega60 = jnp.exp((m20 % 10).astype(jnp.float32)
                      * (-jnp.log(10000.0) / 10.0))
    shift60 = jnp.where(m20 >= 10, jnp.float32(jnp.pi / 2), 0.0)
    rel60 = jnp.concatenate(
        [jnp.broadcast_to(rel[:, i:i + 1], (rows, 20)) for i in range(3)],
        axis=1)
    emb = jnp.sin(rel60 * omega60 + shift60)
    h = jnp.maximum(
        jnp.dot(emb.astype(bf), W1_ref[...].astype(bf),
                preferred_element_type=jnp.float32) + b1_ref[...], 0.0)
    o = jnp.dot(h.astype(bf), W2_ref[...].astype(bf),
                preferred_element_type=jnp.float32) + b2_ref[...]
    gf = f + o
    g3 = gf.reshape(TMC, K, 64)
    w = jnp.sum(g3 * g3[:, 0:1, :], axis=-1)
    mx = jnp.max(w, axis=-1, keepdims=True)
    e = jnp.exp(w - mx)
    sm = e / jnp.sum(e, axis=-1, keepdims=True) * 0.4
    kio = lax.broadcasted_iota(jnp.int32, (TMC, K), 1)
    out_ref[0] = sm + jnp.where(kio == 0, 0.6, 0.0)


def _mlp_call(g, qrep, pW, pb, W1, b1, W2, b2):
    B = g.shape[0]
    MK = g.shape[1]
    M = MK // K
    grid = (B, M // TMC)
    return pl.pallas_call(
        _mlp_body,
        grid=grid,
        in_specs=[
            pl.BlockSpec((1, TMC * K, D_TAB), lambda b, m: (b, m, 0)),
            pl.BlockSpec((1, TMC * K, 3), lambda b, m: (b, m, 0)),
            pl.BlockSpec((64, 64), lambda b, m: (0, 0)),
            pl.BlockSpec((1, 64), lambda b, m: (0, 0)),
            pl.BlockSpec((60, 64), lambda b, m: (0, 0)),
            pl.BlockSpec((1, 64), lambda b, m: (0, 0)),
            pl.BlockSpec((64, 64), lambda b, m: (0, 0)),
            pl.BlockSpec((1, 64), lambda b, m: (0, 0)),
        ],
        out_specs=pl.BlockSpec((1, TMC, K), lambda b, m: (b, m, 0)),
        out_shape=jax.ShapeDtypeStruct((B, M, K), jnp.float32),
    )(g, qrep, pW, pb, W1, b1, W2, b2)


def kernel(feature1, xyz1, query_xyz, error, shift_chose, knn_num,
           prefix_W, prefix_b, pos_W1, pos_b1, pos_W2, pos_b2):
    B, N, FD = feature1.shape
    M = query_xyz.shape[1]
    xyzT = jnp.transpose(xyz1, (0, 2, 1))
    point_index, flat_idx = _knn_call(xyzT, query_xyz)
    table = jnp.concatenate(
        [feature1, xyz1, jnp.zeros((B, N, D_TAB - FD - 3), jnp.float32)],
        axis=-1).reshape(B * N, D_TAB)
    gathered = _sc_gather(table, flat_idx.reshape(-1))
    g = gathered.reshape(B, M * K, D_TAB)
    qrep = jnp.broadcast_to(query_xyz[:, :, None, :],
                            (B, M, K, 3)).reshape(B, M * K, 3)
    weight = _mlp_call(g, qrep, prefix_W, prefix_b.reshape(1, -1),
                       pos_W1, pos_b1.reshape(1, -1),
                       pos_W2, pos_b2.reshape(1, -1))
    return (weight, point_index)


# speedup vs baseline: 14.8939x; 1.0306x over previous
import functools

import jax
import jax.numpy as jnp
from jax import lax
from jax.experimental import pallas as pl
from jax.experimental.pallas import tpu as pltpu
from jax.experimental.pallas import tpu_sc as plsc

K = 16
DEPTH = 6
TMK = 64
TMC = 256
D_TAB = 128


def _knn_body(xyzT_ref, q_ref, idx_ref, flat_ref):
    b = pl.program_id(0)
    n = xyzT_ref.shape[2]
    x = xyzT_ref[0]
    q = q_ref[0]
    xb = x.astype(jnp.bfloat16)
    qb = q.astype(jnp.bfloat16)
    x2 = jnp.sum(x * x, axis=0, keepdims=True)
    q2 = jnp.sum(q * q, axis=1, keepdims=True)
    lane = lax.broadcasted_iota(jnp.int32, (TMK, 128), 1)
    IMAX = jnp.int32(0x7FFFFFFF)
    sstate = [jnp.full((TMK, 128), IMAX, jnp.int32) for _ in range(DEPTH)]
    istate = [jnp.zeros((TMK, 128), jnp.int32) for _ in range(DEPTH)]
    for c in range(n // 128):
        sl = slice(c * 128, (c + 1) * 128)
        dots = jnp.dot(qb, xb[:, sl], preferred_element_type=jnp.float32)
        d = (q2 + x2[:, sl]) - 2.0 * dots
        u = lax.bitcast_convert_type(d, jnp.int32)
        sv = u ^ (lax.shift_right_arithmetic(u, 31) & IMAX)
        iv = lane + c * 128
        for i in range(DEPTH):
            cm = sv < sstate[i]
            ns = jnp.minimum(sv, sstate[i])
            sv = jnp.maximum(sv, sstate[i])
            ni = jnp.where(cm, iv, istate[i])
            iv = jnp.where(cm, istate[i], iv)
            sstate[i], istate[i] = ns, ni
    scat = jnp.concatenate(sstate, axis=1)
    icat = jnp.concatenate(istate, axis=1)
    last_s = jnp.full((TMK, 1), jnp.int32(-0x80000000), jnp.int32)
    last_i = jnp.full((TMK, 1), jnp.int32(-1), jnp.int32)
    for k in range(K):
        valid = (scat > last_s) | ((scat == last_s) & (icat > last_i))
        smin = jnp.min(jnp.where(valid, scat, IMAX), axis=1, keepdims=True)
        icand = jnp.where(valid & (scat == smin), icat, n)
        imin = jnp.min(icand, axis=1, keepdims=True)
        idx_ref[0, :, k:k + 1] = imin
        flat_ref[0, :, k:k + 1] = imin + b * n
        last_s, last_i = smin, imin


def _knn_call(xyzT, query):
    B, _, N = xyzT.shape
    M = query.shape[1]
    grid = (B, M // TMK)
    return pl.pallas_call(
        _knn_body,
        grid=grid,
        in_specs=[
            pl.BlockSpec((1, 3, N), lambda b, m: (b, 0, 0)),
            pl.BlockSpec((1, TMK, 3), lambda b, m: (b, m, 0)),
        ],
        out_specs=[
            pl.BlockSpec((1, TMK, K), lambda b, m: (b, m, 0)),
            pl.BlockSpec((1, TMK, K), lambda b, m: (b, m, 0)),
        ],
        out_shape=[
            jax.ShapeDtypeStruct((B, M, K), jnp.int32),
            jax.ShapeDtypeStruct((B, M, K), jnp.int32),
        ],
    )(xyzT, query)


def _sc_gather(table, idx_flat):
    R = idx_flat.shape[0]
    info = plsc.get_sparse_core_info()
    NW = info.num_cores * info.num_subcores
    rows_per_w = R // NW
    CH = min(512, rows_per_w)
    mesh = plsc.VectorSubcoreMesh(core_axis_name="c", subcore_axis_name="s")

    @functools.partial(
        pl.kernel,
        mesh=mesh,
        out_type=jax.ShapeDtypeStruct((R, D_TAB), jnp.float32),
        scratch_types=[
            pltpu.VMEM((CH,), jnp.int32),
            pltpu.VMEM((CH, D_TAB), jnp.float32),
            pltpu.SemaphoreType.DMA,
        ],
    )
    def gk(table_hbm, idx_hbm, out_hbm, idx_v, rows_v, sem):
        wid = lax.axis_index("s") * info.num_cores + lax.axis_index("c")
        for c in range(rows_per_w // CH):
            base = wid * rows_per_w + c * CH
            pltpu.sync_copy(idx_hbm.at[pl.ds(base, CH)], idx_v)
            pltpu.async_copy(table_hbm.at[idx_v], rows_v, sem).wait()
            pltpu.sync_copy(rows_v, out_hbm.at[pl.ds(base, CH)])

    return gk(table, idx_flat)


def _mlp_body(g_ref, qrep_ref, pW_ref, pb_ref, W1_ref, b1_ref, W2_ref, b2_ref,
              out_ref):
    bf = jnp.bfloat16
    g = g_ref[0]
    feat = g[:, :64]
    rel = g[:, 64:67] - qrep_ref[0]
    f = jnp.dot(feat.astype(bf), pW_ref[...].astype(bf),
                preferred_element_type=jnp.float32) + pb_ref[...]
    rows = rel.shape[0]
    lane60 = lax.broadcasted_iota(jnp.int32, (1, 60), 1)
    m20 = lane60 % 20
    omega60 = jnp.exp((m20 % 10).astype(jnp.float32)
                      * (-jnp.log(10000.0) / 10.0))
    shift60 = jnp.where(m20 >= 10, jnp.float32(jnp.pi / 2), 0.0)
    rel60 = jnp.concatenate(
        [jnp.broadcast_to(rel[:, i:i + 1], (rows, 20)) for i in range(3)],
        axis=1)
    emb = jnp.sin(rel60 * omega60 + shift60)
    h = jnp.maximum(
        jnp.dot(emb.astype(bf), W1_ref[...].astype(bf),
                preferred_element_type=jnp.float32) + b1_ref[...], 0.0)
    o = jnp.dot(h.astype(bf), W2_ref[...].astype(bf),
                preferred_element_type=jnp.float32) + b2_ref[...]
    gf = f + o
    g3 = gf.reshape(TMC, K, 64)
    w = jnp.sum(g3 * g3[:, 0:1, :], axis=-1)
    mx = jnp.max(w, axis=-1, keepdims=True)
    e = jnp.exp(w - mx)
    sm = e / jnp.sum(e, axis=-1, keepdims=True) * 0.4
    kio = lax.broadcasted_iota(jnp.int32, (TMC, K), 1)
    out_ref[0] = sm + jnp.where(kio == 0, 0.6, 0.0)


def _mlp_call(g, qrep, pW, pb, W1, b1, W2, b2):
    B = g.shape[0]
    MK = g.shape[1]
    M = MK // K
    grid = (B, M // TMC)
    return pl.pallas_call(
        _mlp_body,
        grid=grid,
        in_specs=[
            pl.BlockSpec((1, TMC * K, D_TAB), lambda b, m: (b, m, 0)),
            pl.BlockSpec((1, TMC * K, 3), lambda b, m: (b, m, 0)),
            pl.BlockSpec((64, 64), lambda b, m: (0, 0)),
            pl.BlockSpec((1, 64), lambda b, m: (0, 0)),
            pl.BlockSpec((60, 64), lambda b, m: (0, 0)),
            pl.BlockSpec((1, 64), lambda b, m: (0, 0)),
            pl.BlockSpec((64, 64), lambda b, m: (0, 0)),
            pl.BlockSpec((1, 64), lambda b, m: (0, 0)),
        ],
        out_specs=pl.BlockSpec((1, TMC, K), lambda b, m: (b, m, 0)),
        out_shape=jax.ShapeDtypeStruct((B, M, K), jnp.float32),
    )(g, qrep, pW, pb, W1, b1, W2, b2)


def kernel(feature1, xyz1, query_xyz, error, shift_chose, knn_num,
           prefix_W, prefix_b, pos_W1, pos_b1, pos_W2, pos_b2):
    B, N, FD = feature1.shape
    M = query_xyz.shape[1]
    xyzT = jnp.transpose(xyz1, (0, 2, 1))
    point_index, flat_idx = _knn_call(xyzT, query_xyz)
    table = jnp.concatenate(
        [feature1, xyz1, jnp.zeros((B, N, D_TAB - FD - 3), jnp.float32)],
        axis=-1).reshape(B * N, D_TAB)
    gathered = _sc_gather(table, flat_idx.reshape(-1))
    g = gathered.reshape(B, M * K, D_TAB)
    qrep = jnp.broadcast_to(query_xyz[:, :, None, :],
                            (B, M, K, 3)).reshape(B, M * K, 3)
    weight = _mlp_call(g, qrep, prefix_W, prefix_b.reshape(1, -1),
                       pos_W1, pos_b1.reshape(1, -1),
                       pos_W2, pos_b2.reshape(1, -1))
    return (weight, point_index)
